# baseline (speedup 1.0000x reference)
import functools

import jax
import jax.numpy as jnp
from jax import lax
from jax.experimental import pallas as pl
from jax.experimental.pallas import tpu as pltpu
from jax.experimental.pallas import tpu_sc as plsc

N_NODES = 10000
D = 128
E = 320000

NC, NS = 2, 16
NW = NC * NS
NP = 10112
RPT = NP // NS
K = 128
EPT = 10240
R = EPT // K
G = 16
NG = R // G

RB = 632
GRID = NP // RB

_sc_mesh = plsc.VectorSubcoreMesh(core_axis_name="c", subcore_axis_name="s")


@functools.partial(
    pl.kernel,
    out_type=jax.ShapeDtypeStruct((NC, NP, D), jnp.float32),
    mesh=_sc_mesh,
    scratch_types=(
        [pltpu.VMEM((R, K), jnp.int32)]
        + [pltpu.VMEM((G, K), jnp.int32)]
        + [pltpu.VMEM((K, D), jnp.float32)] * 2
        + [pltpu.VMEM_SHARED((NP, D), jnp.float32)]
        + [pltpu.SemaphoreType.DMA] * 2
        + [pltpu.SemaphoreType.DMA]
    ),
)
def _edge_pass(g_hbm, src_hbm, dst_hbm, zeros_hbm, out_hbm,
               dst_v, sbuf, rw0, rw1, acc, gs0, gs1, ssem):
    rows = (rw0, rw1)
    gsem = (gs0, gs1)
    cid = lax.axis_index("c")
    sid = lax.axis_index("s")
    w = cid * NS + sid
    r0 = sid * RPT
    pltpu.sync_copy(zeros_hbm.at[pl.ds(r0, RPT)], acc.at[pl.ds(r0, RPT)])
    pltpu.sync_copy(dst_hbm.at[w], dst_v)
    plsc.subcore_barrier()

    def group(g, carry):
        pltpu.sync_copy(src_hbm.at[w].at[g], sbuf)
        c0 = g * G
        descs = [None, None]
        descs[0] = pltpu.async_copy(g_hbm.at[sbuf.at[0]], rows[0], gsem[0])
        descs[1] = pltpu.async_copy(g_hbm.at[sbuf.at[1]], rows[1], gsem[1])
        for j in range(G):
            b = j % 2
            descs[b].wait()
            pltpu.async_copy(rows[b], acc.at[dst_v.at[c0 + j]], ssem,
                             add=True).wait()
            if j < G - 2:
                descs[b] = pltpu.async_copy(
                    g_hbm.at[sbuf.at[j + 2]], rows[b], gsem[b])
        return carry

    lax.fori_loop(0, NG, group, 0)
    plsc.subcore_barrier()
    pltpu.sync_copy(acc.at[pl.ds(r0, RPT)], out_hbm.at[cid].at[pl.ds(r0, RPT)])


@functools.partial(
    pl.kernel,
    out_type=jax.ShapeDtypeStruct((NC, NP, 16), jnp.float32),
    mesh=_sc_mesh,
    scratch_types=[
        pltpu.VMEM((R, K), jnp.int32),
        pltpu.VMEM((K, 16), jnp.float32),
        pltpu.VMEM_SHARED((NP, 16), jnp.float32),
    ],
)
def _deg_pass(dst_hbm, zeros_hbm, out_hbm, dst_v, ones_v, acc):
    cid = lax.axis_index("c")
    sid = lax.axis_index("s")
    w = cid * NS + sid
    r0 = sid * RPT
    pltpu.sync_copy(zeros_hbm.at[pl.ds(r0, RPT)], acc.at[pl.ds(r0, RPT)])
    pltpu.sync_copy(dst_hbm.at[w], dst_v)

    def fill(i, carry):
        ones_v[i] = jnp.ones((16,), jnp.float32)
        return carry

    lax.fori_loop(0, K, fill, 0)
    plsc.subcore_barrier()

    def body(c, carry):
        pltpu.sync_copy(ones_v, acc.at[dst_v.at[c]], add=True)
        return carry

    lax.fori_loop(0, R, body, 0)
    plsc.subcore_barrier()
    pltpu.sync_copy(acc.at[pl.ds(r0, RPT)], out_hbm.at[cid].at[pl.ds(r0, RPT)])


def _dis_from_counts(dc_ref):
    deg = dc_ref[0, :, 0] + dc_ref[1, :, 0] + 1.0
    return lax.rsqrt(deg)


def _tc1_body(x_ref, w1_ref, dc_ref, g1_ref):
    dis = _dis_from_counts(dc_ref)
    h = jnp.dot(x_ref[...], w1_ref[...], preferred_element_type=jnp.float32)
    g1_ref[...] = h * dis[:, None]


def _tc2_body(p_ref, g1_ref, dc_ref, b1_ref, w2_ref, g2_ref):
    dis = _dis_from_counts(dc_ref)
    s = (p_ref[0] + p_ref[1] + g1_ref[...]) * dis[:, None] + b1_ref[...]
    h = jnp.maximum(s, 0.0)
    g2_ref[...] = jnp.dot(h, w2_ref[...],
                          preferred_element_type=jnp.float32) * dis[:, None]


def _tc3_body(p_ref, g2_ref, dc_ref, b2_ref, o_ref):
    dis = _dis_from_counts(dc_ref)
    z = (p_ref[0] + p_ref[1] + g2_ref[...]) * dis[:, None] + b2_ref[...]
    m = jnp.max(z, axis=-1, keepdims=True)
    lse = jnp.log(jnp.sum(jnp.exp(z - m), axis=-1, keepdims=True)) + m
    o_ref[...] = z - lse


_row_spec = pl.BlockSpec((RB, D), lambda i: (i, 0))
_dc_spec = pl.BlockSpec((2, RB, 16), lambda i: (0, i, 0))
_w_spec = pl.BlockSpec((D, D), lambda i: (0, 0))
_b_spec = pl.BlockSpec((1, D), lambda i: (0, 0))
_p_spec = pl.BlockSpec((2, RB, D), lambda i: (0, i, 0))
_out_f32 = jax.ShapeDtypeStruct((NP, D), jnp.float32)

_tc1 = pl.pallas_call(
    _tc1_body, grid=(GRID,),
    in_specs=[_row_spec, _w_spec, _dc_spec],
    out_specs=_row_spec, out_shape=_out_f32)

_tc2 = pl.pallas_call(
    _tc2_body, grid=(GRID,),
    in_specs=[_p_spec, _row_spec, _dc_spec, _b_spec, _w_spec],
    out_specs=_row_spec, out_shape=_out_f32)

_tc3 = pl.pallas_call(
    _tc3_body, grid=(GRID,),
    in_specs=[_p_spec, _row_spec, _dc_spec, _b_spec],
    out_specs=_row_spec, out_shape=_out_f32)


def _pad_edges(idx):
    per_tile = idx.astype(jnp.int32).reshape(NW, E // NW)
    padded = jnp.pad(per_tile, ((0, 0), (0, EPT - E // NW)),
                     constant_values=NP - 1)
    return padded.reshape(NW, NG, G, K)


def kernel(x, edge_index, new_edge_indexs, W1, b1, W2, b2):
    src = _pad_edges(edge_index[0])
    dst = _pad_edges(edge_index[1]).reshape(NW, R, K)
    xp = jnp.zeros((NP, D), jnp.float32).at[:N_NODES].set(x)
    zeros_full = jnp.zeros((NP, D), jnp.float32)
    zeros_deg = jnp.zeros((NP, 16), jnp.float32)

    dc = _deg_pass(dst, zeros_deg)
    g1 = _tc1(xp, W1, dc)
    p1 = _edge_pass(g1, src, dst, zeros_full)
    g2 = _tc2(p1, g1, dc, b1.reshape(1, D), W2)
    p2 = _edge_pass(g2, src, dst, zeros_full)
    out = _tc3(p2, g2, dc, b2.reshape(1, D))
    return out[:N_NODES]


# speedup vs baseline: 2.4836x; 2.4836x over previous
import functools

import jax
import jax.numpy as jnp
from jax import lax
from jax.experimental import pallas as pl
from jax.experimental.pallas import tpu as pltpu
from jax.experimental.pallas import tpu_sc as plsc

N_NODES = 10000
D = 128
E = 320000

NC, NS = 2, 16
NW = NC * NS
NP = 10112
RPT = NP // NS
K = 80
EPT = 10000
R = EPT // K
G = 25
NG = R // G

RB = 632
GRID = NP // RB

_sc_mesh = plsc.VectorSubcoreMesh(core_axis_name="c", subcore_axis_name="s")


@functools.partial(
    pl.kernel,
    out_type=jax.ShapeDtypeStruct((NC, NP, D), jnp.float32),
    mesh=_sc_mesh,
    scratch_types=(
        [pltpu.VMEM((R, K), jnp.int32)]
        + [pltpu.VMEM((G, K), jnp.int32)]
        + [pltpu.VMEM((K, D), jnp.float32)] * 2
        + [pltpu.VMEM_SHARED((NP, D), jnp.float32)]
        + [pltpu.SemaphoreType.DMA] * 2
        + [pltpu.SemaphoreType.DMA]
    ),
)
def _edge_pass(g_hbm, src_hbm, dst_hbm, zeros_hbm, out_hbm,
               dst_v, sbuf, rw0, rw1, acc, gs0, gs1, ssem):
    rows = (rw0, rw1)
    gsem = (gs0, gs1)
    cid = lax.axis_index("c")
    sid = lax.axis_index("s")
    w = cid * NS + sid
    r0 = sid * RPT
    pltpu.sync_copy(zeros_hbm.at[pl.ds(r0, RPT)], acc.at[pl.ds(r0, RPT)])
    pltpu.sync_copy(dst_hbm.at[w], dst_v)
    plsc.subcore_barrier()

    def group(g, carry):
        pltpu.sync_copy(src_hbm.at[w].at[g], sbuf)
        c0 = g * G
        descs = [None, None]
        descs[0] = pltpu.async_copy(g_hbm.at[sbuf.at[0]], rows[0], gsem[0])
        descs[1] = pltpu.async_copy(g_hbm.at[sbuf.at[1]], rows[1], gsem[1])
        for j in range(G):
            b = j % 2
            descs[b].wait()
            pltpu.async_copy(rows[b], acc.at[dst_v.at[c0 + j]], ssem,
                             add=True).wait()
            if j < G - 2:
                descs[b] = pltpu.async_copy(
                    g_hbm.at[sbuf.at[j + 2]], rows[b], gsem[b])
        return carry

    lax.fori_loop(0, NG, group, 0)
    plsc.subcore_barrier()
    pltpu.sync_copy(acc.at[pl.ds(r0, RPT)], out_hbm.at[cid].at[pl.ds(r0, RPT)])


@functools.partial(
    pl.kernel,
    out_type=jax.ShapeDtypeStruct((NC, NP, 16), jnp.float32),
    mesh=_sc_mesh,
    scratch_types=[
        pltpu.VMEM((R, K), jnp.int32),
        pltpu.VMEM((K, 16), jnp.float32),
        pltpu.VMEM_SHARED((NP, 16), jnp.float32),
    ],
)
def _deg_pass(dst_hbm, zeros_hbm, out_hbm, dst_v, ones_v, acc):
    cid = lax.axis_index("c")
    sid = lax.axis_index("s")
    w = cid * NS + sid
    r0 = sid * RPT
    pltpu.sync_copy(zeros_hbm.at[pl.ds(r0, RPT)], acc.at[pl.ds(r0, RPT)])
    pltpu.sync_copy(dst_hbm.at[w], dst_v)

    def fill(i, carry):
        ones_v[i] = jnp.ones((16,), jnp.float32)
        return carry

    lax.fori_loop(0, K, fill, 0)
    plsc.subcore_barrier()

    def body(c, carry):
        pltpu.sync_copy(ones_v, acc.at[dst_v.at[c]], add=True)
        return carry

    lax.fori_loop(0, R, body, 0)
    plsc.subcore_barrier()
    pltpu.sync_copy(acc.at[pl.ds(r0, RPT)], out_hbm.at[cid].at[pl.ds(r0, RPT)])


def _dis_from_counts(dc_ref):
    deg = dc_ref[0, :, 0] + dc_ref[1, :, 0] + 1.0
    return lax.rsqrt(deg)


def _tc1_body(x_ref, w1_ref, dc_ref, g1_ref):
    dis = _dis_from_counts(dc_ref)
    h = jnp.dot(x_ref[...], w1_ref[...], preferred_element_type=jnp.float32)
    g1_ref[...] = h * dis[:, None]


def _tc2_body(p_ref, g1_ref, dc_ref, b1_ref, w2_ref, g2_ref):
    dis = _dis_from_counts(dc_ref)
    s = (p_ref[0] + p_ref[1] + g1_ref[...]) * dis[:, None] + b1_ref[...]
    h = jnp.maximum(s, 0.0)
    g2_ref[...] = jnp.dot(h, w2_ref[...],
                          preferred_element_type=jnp.float32) * dis[:, None]


def _tc3_body(p_ref, g2_ref, dc_ref, b2_ref, o_ref):
    dis = _dis_from_counts(dc_ref)
    z = (p_ref[0] + p_ref[1] + g2_ref[...]) * dis[:, None] + b2_ref[...]
    m = jnp.max(z, axis=-1, keepdims=True)
    lse = jnp.log(jnp.sum(jnp.exp(z - m), axis=-1, keepdims=True)) + m
    o_ref[...] = z - lse


_row_spec = pl.BlockSpec((RB, D), lambda i: (i, 0))
_dc_spec = pl.BlockSpec((2, RB, 16), lambda i: (0, i, 0))
_w_spec = pl.BlockSpec((D, D), lambda i: (0, 0))
_b_spec = pl.BlockSpec((1, D), lambda i: (0, 0))
_p_spec = pl.BlockSpec((2, RB, D), lambda i: (0, i, 0))
_out_f32 = jax.ShapeDtypeStruct((NP, D), jnp.float32)

_tc1 = pl.pallas_call(
    _tc1_body, grid=(GRID,),
    in_specs=[_row_spec, _w_spec, _dc_spec],
    out_specs=_row_spec, out_shape=_out_f32)

_tc2 = pl.pallas_call(
    _tc2_body, grid=(GRID,),
    in_specs=[_p_spec, _row_spec, _dc_spec, _b_spec, _w_spec],
    out_specs=_row_spec, out_shape=_out_f32)

_tc3 = pl.pallas_call(
    _tc3_body, grid=(GRID,),
    in_specs=[_p_spec, _row_spec, _dc_spec, _b_spec],
    out_specs=_row_spec, out_shape=_out_f32)


def _pad_edges(idx):
    per_tile = idx.astype(jnp.int32).reshape(NW, E // NW)
    if EPT > E // NW:
        per_tile = jnp.pad(per_tile, ((0, 0), (0, EPT - E // NW)),
                           constant_values=NP - 1)
    return per_tile.reshape(NW, NG, G, K)


def kernel(x, edge_index, new_edge_indexs, W1, b1, W2, b2):
    src = _pad_edges(edge_index[0])
    dst = _pad_edges(edge_index[1]).reshape(NW, R, K)
    xp = jnp.zeros((NP, D), jnp.float32).at[:N_NODES].set(x)
    zeros_full = jnp.zeros((NP, D), jnp.float32)
    zeros_deg = jnp.zeros((NP, 16), jnp.float32)

    dc = _deg_pass(dst, zeros_deg)
    g1 = _tc1(xp, W1, dc)
    p1 = _edge_pass(g1, src, dst, zeros_full)
    g2 = _tc2(p1, g1, dc, b1.reshape(1, D), W2)
    p2 = _edge_pass(g2, src, dst, zeros_full)
    out = _tc3(p2, g2, dc, b2.reshape(1, D))
    return out[:N_NODES]


# speedup vs baseline: 2.7225x; 1.0962x over previous
import functools

import jax
import jax.numpy as jnp
from jax import lax
from jax.experimental import pallas as pl
from jax.experimental.pallas import tpu as pltpu
from jax.experimental.pallas import tpu_sc as plsc

N_NODES = 10000
D = 128
E = 320000

NC, NS = 2, 16
NW = NC * NS
NP = 10112
RPT = NP // NS
K = 80
EPT = 10000
R = EPT // K
G = 25
NG = R // G
NBUF = 3

RB = 632
GRID = NP // RB

_sc_mesh = plsc.VectorSubcoreMesh(core_axis_name="c", subcore_axis_name="s")


@functools.partial(
    pl.kernel,
    out_type=jax.ShapeDtypeStruct((NC, NP, D), jnp.float32),
    mesh=_sc_mesh,
    scratch_types=(
        [pltpu.VMEM((G, K), jnp.int32)]
        + [pltpu.VMEM((G, K), jnp.int32)]
        + [pltpu.VMEM((K, D), jnp.float32)] * NBUF
        + [pltpu.VMEM_SHARED((NP, D), jnp.float32)]
        + [pltpu.SemaphoreType.DMA] * NBUF
        + [pltpu.SemaphoreType.DMA]
    ),
)
def _edge_pass(g_hbm, src_hbm, dst_hbm, zeros_hbm, out_hbm,
               dst_v, sbuf, *rest):
    rows = rest[:NBUF]
    acc = rest[NBUF]
    gsem = rest[NBUF + 1:2 * NBUF + 1]
    ssem = rest[2 * NBUF + 1]
    cid = lax.axis_index("c")
    sid = lax.axis_index("s")
    w = cid * NS + sid
    r0 = sid * RPT
    pltpu.sync_copy(zeros_hbm.at[pl.ds(r0, RPT)], acc.at[pl.ds(r0, RPT)])
    plsc.subcore_barrier()

    def group(g, carry):
        pltpu.sync_copy(src_hbm.at[w].at[g], sbuf)
        pltpu.sync_copy(dst_hbm.at[w].at[g], dst_v)
        descs = [
            pltpu.async_copy(g_hbm.at[sbuf.at[b]], rows[b], gsem[b])
            for b in range(NBUF)
        ]
        for j in range(G):
            b = j % NBUF
            descs[b].wait()
            pltpu.async_copy(rows[b], acc.at[dst_v.at[j]], ssem,
                             add=True).wait()
            if j < G - NBUF:
                descs[b] = pltpu.async_copy(
                    g_hbm.at[sbuf.at[j + NBUF]], rows[b], gsem[b])
        return carry

    lax.fori_loop(0, NG, group, 0)
    plsc.subcore_barrier()
    pltpu.sync_copy(acc.at[pl.ds(r0, RPT)], out_hbm.at[cid].at[pl.ds(r0, RPT)])


@functools.partial(
    pl.kernel,
    out_type=jax.ShapeDtypeStruct((NC, NP, 16), jnp.float32),
    mesh=_sc_mesh,
    scratch_types=[
        pltpu.VMEM((R, K), jnp.int32),
        pltpu.VMEM((K, 16), jnp.float32),
        pltpu.VMEM_SHARED((NP, 16), jnp.float32),
    ],
)
def _deg_pass(dst_hbm, zeros_hbm, out_hbm, dst_v, ones_v, acc):
    cid = lax.axis_index("c")
    sid = lax.axis_index("s")
    w = cid * NS + sid
    r0 = sid * RPT
    pltpu.sync_copy(zeros_hbm.at[pl.ds(r0, RPT)], acc.at[pl.ds(r0, RPT)])
    pltpu.sync_copy(dst_hbm.at[w], dst_v)

    def fill(i, carry):
        ones_v[i] = jnp.ones((16,), jnp.float32)
        return carry

    lax.fori_loop(0, K, fill, 0)
    plsc.subcore_barrier()

    def body(c, carry):
        pltpu.sync_copy(ones_v, acc.at[dst_v.at[c]], add=True)
        return carry

    lax.fori_loop(0, R, body, 0)
    plsc.subcore_barrier()
    pltpu.sync_copy(acc.at[pl.ds(r0, RPT)], out_hbm.at[cid].at[pl.ds(r0, RPT)])


def _dis_from_counts(dc_ref):
    deg = dc_ref[0, :, 0] + dc_ref[1, :, 0] + 1.0
    return lax.rsqrt(deg)


def _tc1_body(x_ref, w1_ref, dc_ref, g1_ref):
    dis = _dis_from_counts(dc_ref)
    h = jnp.dot(x_ref[...], w1_ref[...], preferred_element_type=jnp.float32)
    g1_ref[...] = h * dis[:, None]


def _tc2_body(p_ref, g1_ref, dc_ref, b1_ref, w2_ref, g2_ref):
    dis = _dis_from_counts(dc_ref)
    s = (p_ref[0] + p_ref[1] + g1_ref[...]) * dis[:, None] + b1_ref[...]
    h = jnp.maximum(s, 0.0)
    g2_ref[...] = jnp.dot(h, w2_ref[...],
                          preferred_element_type=jnp.float32) * dis[:, None]


def _tc3_body(p_ref, g2_ref, dc_ref, b2_ref, o_ref):
    dis = _dis_from_counts(dc_ref)
    z = (p_ref[0] + p_ref[1] + g2_ref[...]) * dis[:, None] + b2_ref[...]
    m = jnp.max(z, axis=-1, keepdims=True)
    lse = jnp.log(jnp.sum(jnp.exp(z - m), axis=-1, keepdims=True)) + m
    o_ref[...] = z - lse


_row_spec = pl.BlockSpec((RB, D), lambda i: (i, 0))
_dc_spec = pl.BlockSpec((2, RB, 16), lambda i: (0, i, 0))
_w_spec = pl.BlockSpec((D, D), lambda i: (0, 0))
_b_spec = pl.BlockSpec((1, D), lambda i: (0, 0))
_p_spec = pl.BlockSpec((2, RB, D), lambda i: (0, i, 0))
_out_f32 = jax.ShapeDtypeStruct((NP, D), jnp.float32)

_tc1 = pl.pallas_call(
    _tc1_body, grid=(GRID,),
    in_specs=[_row_spec, _w_spec, _dc_spec],
    out_specs=_row_spec, out_shape=_out_f32)

_tc2 = pl.pallas_call(
    _tc2_body, grid=(GRID,),
    in_specs=[_p_spec, _row_spec, _dc_spec, _b_spec, _w_spec],
    out_specs=_row_spec, out_shape=_out_f32)

_tc3 = pl.pallas_call(
    _tc3_body, grid=(GRID,),
    in_specs=[_p_spec, _row_spec, _dc_spec, _b_spec],
    out_specs=_row_spec, out_shape=_out_f32)


def _pad_edges(idx):
    per_tile = idx.astype(jnp.int32).reshape(NW, E // NW)
    if EPT > E // NW:
        per_tile = jnp.pad(per_tile, ((0, 0), (0, EPT - E // NW)),
                           constant_values=NP - 1)
    return per_tile.reshape(NW, NG, G, K)


def kernel(x, edge_index, new_edge_indexs, W1, b1, W2, b2):
    src = _pad_edges(edge_index[0])
    dst = _pad_edges(edge_index[1])
    dst_flat = dst.reshape(NW, R, K)
    xp = jnp.zeros((NP, D), jnp.float32).at[:N_NODES].set(x)
    zeros_full = jnp.zeros((NP, D), jnp.float32)
    zeros_deg = jnp.zeros((NP, 16), jnp.float32)

    dc = _deg_pass(dst_flat, zeros_deg)
    g1 = _tc1(xp, W1, dc)
    p1 = _edge_pass(g1, src, dst, zeros_full)
    g2 = _tc2(p1, g1, dc, b1.reshape(1, D), W2)
    p2 = _edge_pass(g2, src, dst, zeros_full)
    out = _tc3(p2, g2, dc, b2.reshape(1, D))
    return out[:N_NODES]
